# initial kernel scaffold (unmeasured)
import jax
import jax.numpy as jnp
from jax import lax
from jax.experimental import pallas as pl
from jax.experimental.pallas import tpu as pltpu


def kernel(Q, K, V):
    b, q_len, h, d = Q.shape
    k_len = K.shape[1]
    scale = d ** -0.5

    def partials_body(q_ref, k_ref, v_ref, m_ref, l_ref, o_ref):
        q = q_ref[0, 0]
        k = k_ref[0, :, 0, :]
        v = v_ref[0, :, 0, :]
        s = lax.dot_general(
            q, k, (((1,), (1,)), ((), ())),
            preferred_element_type=jnp.float32,
        ) * scale
        m = jnp.max(s, axis=1, keepdims=True)
        p = jnp.exp(s - m)
        l = jnp.sum(p, axis=1, keepdims=True)
        o = lax.dot_general(
            p, v, (((1,), (0,)), ((), ())),
            preferred_element_type=jnp.float32,
        )
        m_ref[...] = m
        l_ref[...] = l
        o_ref[...] = o.reshape(1, 1, d)

    m_part, l_part, o_part = pl.pallas_call(
        partials_body,
        grid=(b, h),
        in_specs=[
            pl.BlockSpec((1, 1, 1, d), lambda bi, hi: (bi, 0, hi, 0)),
            pl.BlockSpec((1, k_len, 1, d), lambda bi, hi: (bi, 0, hi, 0)),
            pl.BlockSpec((1, k_len, 1, d), lambda bi, hi: (bi, 0, hi, 0)),
        ],
        out_specs=[
            pl.BlockSpec((1, 1), lambda bi, hi: (bi, hi)),
            pl.BlockSpec((1, 1), lambda bi, hi: (bi, hi)),
            pl.BlockSpec((1, 1, d), lambda bi, hi: (bi, hi, 0)),
        ],
        out_shape=[
            jax.ShapeDtypeStruct((b, h), jnp.float32),
            jax.ShapeDtypeStruct((b, h), jnp.float32),
            jax.ShapeDtypeStruct((b, h, d), jnp.float32),
        ],
    )(Q, K, V)

    def combine_body(m_ref, l_ref, o_ref, out_ref,
                     rm_ref, rl_ref, ro_ref, send_sems, recv_sems):
        my_x = lax.axis_index("x")
        my_y = lax.axis_index("y")
        nbr = (1 - my_x, my_y)

        barrier = pltpu.get_barrier_semaphore()
        pl.semaphore_signal(
            barrier, inc=1, device_id=nbr,
            device_id_type=pl.DeviceIdType.MESH,
        )
        pl.semaphore_wait(barrier, 1)

        copies = []
        for i, (src, dst) in enumerate(
            [(m_ref, rm_ref), (l_ref, rl_ref), (o_ref, ro_ref)]
        ):
            c = pltpu.make_async_remote_copy(
                src_ref=src,
                dst_ref=dst,
                send_sem=send_sems.at[i],
                recv_sem=recv_sems.at[i],
                device_id=nbr,
                device_id_type=pl.DeviceIdType.MESH,
            )
            c.start()
            copies.append(c)
        for c in copies:
            c.wait()

        m_loc = m_ref[...]
        m_rem = rm_ref[...]
        m_new = jnp.maximum(m_loc, m_rem)
        a_loc = jnp.exp(m_loc - m_new)
        a_rem = jnp.exp(m_rem - m_new)
        l_new = a_loc * l_ref[...] + a_rem * rl_ref[...]
        o_new = (
            a_loc[:, :, None] * o_ref[...] + a_rem[:, :, None] * ro_ref[...]
        ) / l_new[:, :, None]
        out_ref[:, 0, :, :] = o_new

    return pl.pallas_call(
        combine_body,
        in_specs=[
            pl.BlockSpec(memory_space=pltpu.VMEM),
            pl.BlockSpec(memory_space=pltpu.VMEM),
            pl.BlockSpec(memory_space=pltpu.VMEM),
        ],
        out_specs=pl.BlockSpec(memory_space=pltpu.VMEM),
        out_shape=jax.ShapeDtypeStruct((b, q_len, h, d), jnp.float32),
        scratch_shapes=[
            pltpu.VMEM((b, h), jnp.float32),
            pltpu.VMEM((b, h), jnp.float32),
            pltpu.VMEM((b, h, d), jnp.float32),
            pltpu.SemaphoreType.DMA((3,)),
            pltpu.SemaphoreType.DMA((3,)),
        ],
        compiler_params=pltpu.CompilerParams(collective_id=0),
    )(m_part, l_part, o_part)


# baseline (device time: 329334 ns/iter reference)
import jax
import jax.numpy as jnp
from jax import lax
from jax.experimental import pallas as pl
from jax.experimental.pallas import tpu as pltpu


def kernel(Q, K, V):
    b, q_len, h, d = Q.shape
    k_len = K.shape[1]
    scale = d ** -0.5

    def partials_body(q_ref, k_ref, v_ref, m_ref, l_ref, o_ref):
        q = q_ref[0, 0]
        k = k_ref[0]
        v = v_ref[0]
        s = jnp.sum(k * q[None, :, :], axis=-1) * scale
        m = jnp.max(s, axis=0)
        p = jnp.exp(s - m[None, :])
        l = jnp.sum(p, axis=0)
        o = jnp.sum(p[:, :, None] * v, axis=0)
        m_ref[...] = m.reshape(1, 1, h)
        l_ref[...] = l.reshape(1, 1, h)
        o_ref[...] = o.reshape(1, h, d)

    m_part, l_part, o_part = pl.pallas_call(
        partials_body,
        grid=(b,),
        in_specs=[
            pl.BlockSpec((1, 1, h, d), lambda bi: (bi, 0, 0, 0)),
            pl.BlockSpec((1, k_len, h, d), lambda bi: (bi, 0, 0, 0)),
            pl.BlockSpec((1, k_len, h, d), lambda bi: (bi, 0, 0, 0)),
        ],
        out_specs=[
            pl.BlockSpec((1, 1, h), lambda bi: (bi, 0, 0)),
            pl.BlockSpec((1, 1, h), lambda bi: (bi, 0, 0)),
            pl.BlockSpec((1, h, d), lambda bi: (bi, 0, 0)),
        ],
        out_shape=[
            jax.ShapeDtypeStruct((b, 1, h), jnp.float32),
            jax.ShapeDtypeStruct((b, 1, h), jnp.float32),
            jax.ShapeDtypeStruct((b, h, d), jnp.float32),
        ],
        compiler_params=pltpu.CompilerParams(
            vmem_limit_bytes=100 * 1024 * 1024,
        ),
    )(Q, K, V)

    def combine_body(m_ref, l_ref, o_ref, out_ref,
                     rm_ref, rl_ref, ro_ref, send_sems, recv_sems):
        my_x = lax.axis_index("x")
        my_y = lax.axis_index("y")
        nbr = (1 - my_x, my_y)

        barrier = pltpu.get_barrier_semaphore()
        pl.semaphore_signal(
            barrier, inc=1, device_id=nbr,
            device_id_type=pl.DeviceIdType.MESH,
        )
        pl.semaphore_wait(barrier, 1)

        copies = []
        for i, (src, dst) in enumerate(
            [(m_ref, rm_ref), (l_ref, rl_ref), (o_ref, ro_ref)]
        ):
            c = pltpu.make_async_remote_copy(
                src_ref=src,
                dst_ref=dst,
                send_sem=send_sems.at[i],
                recv_sem=recv_sems.at[i],
                device_id=nbr,
                device_id_type=pl.DeviceIdType.MESH,
            )
            c.start()
            copies.append(c)
        for c in copies:
            c.wait()

        m_loc = m_ref[...]
        m_rem = rm_ref[...]
        m_new = jnp.maximum(m_loc, m_rem)
        a_loc = jnp.exp(m_loc - m_new)
        a_rem = jnp.exp(m_rem - m_new)
        l_new = a_loc * l_ref[...] + a_rem * rl_ref[...]
        o_new = (
            a_loc[:, 0, :, None] * o_ref[...]
            + a_rem[:, 0, :, None] * ro_ref[...]
        ) / l_new[:, 0, :, None]
        out_ref[:, 0, :, :] = o_new

    return pl.pallas_call(
        combine_body,
        in_specs=[
            pl.BlockSpec(memory_space=pltpu.VMEM),
            pl.BlockSpec(memory_space=pltpu.VMEM),
            pl.BlockSpec(memory_space=pltpu.VMEM),
        ],
        out_specs=pl.BlockSpec(memory_space=pltpu.VMEM),
        out_shape=jax.ShapeDtypeStruct((b, q_len, h, d), jnp.float32),
        scratch_shapes=[
            pltpu.VMEM((b, 1, h), jnp.float32),
            pltpu.VMEM((b, 1, h), jnp.float32),
            pltpu.VMEM((b, h, d), jnp.float32),
            pltpu.SemaphoreType.DMA((3,)),
            pltpu.SemaphoreType.DMA((3,)),
        ],
        compiler_params=pltpu.CompilerParams(collective_id=0),
    )(m_part, l_part, o_part)


# device time: 190030 ns/iter; 1.7331x vs baseline; 1.7331x over previous
import jax
import jax.numpy as jnp
from jax import lax
from jax.experimental import pallas as pl
from jax.experimental.pallas import tpu as pltpu


def kernel(Q, K, V):
    b, q_len, h, d = Q.shape
    k_len = K.shape[1]
    hd = h * d
    scale = d ** -0.5

    Qf = Q.reshape(b, 1, hd)
    Kf = K.reshape(b, k_len, hd)
    Vf = V.reshape(b, k_len, hd)

    def partials_body(q_ref, k_ref, v_ref, m_ref, l_ref, o_ref):
        qf = q_ref[0]
        k2 = k_ref[0]
        v2 = v_ref[0]

        rows = lax.broadcasted_iota(jnp.int32, (hd, h), 0) // d
        cols = lax.broadcasted_iota(jnp.int32, (hd, h), 1)
        M = (rows == cols).astype(jnp.float32)

        kq = k2 * qf
        s = lax.dot_general(
            kq, M, (((1,), (0,)), ((), ())),
            preferred_element_type=jnp.float32,
        ) * scale
        m = jnp.max(s, axis=0, keepdims=True)
        p = jnp.exp(s - m)
        l = jnp.sum(p, axis=0, keepdims=True)
        pbig = lax.dot_general(
            p, M, (((1,), (1,)), ((), ())),
            preferred_element_type=jnp.float32,
        )
        pv = pbig * v2
        ones = jnp.ones((1, k_len), jnp.float32)
        o = lax.dot_general(
            ones, pv, (((1,), (0,)), ((), ())),
            preferred_element_type=jnp.float32,
        )

        m_ref[...] = m.reshape(1, 1, h)
        l_ref[...] = l.reshape(1, 1, h)
        o_ref[...] = o.reshape(1, 1, hd)

    m_part, l_part, o_part = pl.pallas_call(
        partials_body,
        grid=(b,),
        in_specs=[
            pl.BlockSpec((1, 1, hd), lambda bi: (bi, 0, 0)),
            pl.BlockSpec((1, k_len, hd), lambda bi: (bi, 0, 0)),
            pl.BlockSpec((1, k_len, hd), lambda bi: (bi, 0, 0)),
        ],
        out_specs=[
            pl.BlockSpec((1, 1, h), lambda bi: (bi, 0, 0)),
            pl.BlockSpec((1, 1, h), lambda bi: (bi, 0, 0)),
            pl.BlockSpec((1, 1, hd), lambda bi: (bi, 0, 0)),
        ],
        out_shape=[
            jax.ShapeDtypeStruct((b, 1, h), jnp.float32),
            jax.ShapeDtypeStruct((b, 1, h), jnp.float32),
            jax.ShapeDtypeStruct((b, 1, hd), jnp.float32),
        ],
        compiler_params=pltpu.CompilerParams(
            vmem_limit_bytes=100 * 1024 * 1024,
        ),
    )(Qf, Kf, Vf)

    def combine_body(m_ref, l_ref, o_ref, out_ref,
                     rm_ref, rl_ref, ro_ref, send_sems, recv_sems):
        my_x = lax.axis_index("x")
        my_y = lax.axis_index("y")
        nbr = (1 - my_x, my_y)

        barrier = pltpu.get_barrier_semaphore()
        pl.semaphore_signal(
            barrier, inc=1, device_id=nbr,
            device_id_type=pl.DeviceIdType.MESH,
        )
        pl.semaphore_wait(barrier, 1)

        copies = []
        for i, (src, dst) in enumerate(
            [(m_ref, rm_ref), (l_ref, rl_ref), (o_ref, ro_ref)]
        ):
            c = pltpu.make_async_remote_copy(
                src_ref=src,
                dst_ref=dst,
                send_sem=send_sems.at[i],
                recv_sem=recv_sems.at[i],
                device_id=nbr,
                device_id_type=pl.DeviceIdType.MESH,
            )
            c.start()
            copies.append(c)
        for c in copies:
            c.wait()

        m_loc = m_ref[:, 0, :]
        m_rem = rm_ref[:, 0, :]
        m_new = jnp.maximum(m_loc, m_rem)
        a_loc = jnp.exp(m_loc - m_new)
        a_rem = jnp.exp(m_rem - m_new)
        l_new = a_loc * l_ref[:, 0, :] + a_rem * rl_ref[:, 0, :]

        rows = lax.broadcasted_iota(jnp.int32, (h, hd), 0)
        cols = lax.broadcasted_iota(jnp.int32, (h, hd), 1) // d
        E = (rows == cols).astype(jnp.float32)

        def bcast(x):
            return lax.dot_general(
                x, E, (((1,), (0,)), ((), ())),
                preferred_element_type=jnp.float32,
            )

        o_new = (
            bcast(a_loc) * o_ref[:, 0, :]
            + bcast(a_rem) * ro_ref[:, 0, :]
        ) / bcast(l_new)
        out_ref[:, 0, :] = o_new

    out = pl.pallas_call(
        combine_body,
        in_specs=[
            pl.BlockSpec(memory_space=pltpu.VMEM),
            pl.BlockSpec(memory_space=pltpu.VMEM),
            pl.BlockSpec(memory_space=pltpu.VMEM),
        ],
        out_specs=pl.BlockSpec(memory_space=pltpu.VMEM),
        out_shape=jax.ShapeDtypeStruct((b, 1, hd), jnp.float32),
        scratch_shapes=[
            pltpu.VMEM((b, 1, h), jnp.float32),
            pltpu.VMEM((b, 1, h), jnp.float32),
            pltpu.VMEM((b, 1, hd), jnp.float32),
            pltpu.SemaphoreType.DMA((3,)),
            pltpu.SemaphoreType.DMA((3,)),
        ],
        compiler_params=pltpu.CompilerParams(collective_id=0),
    )(m_part, l_part, o_part)

    return out.reshape(b, q_len, h, d)


# device time: 41892 ns/iter; 7.8615x vs baseline; 4.5362x over previous
import jax
import jax.numpy as jnp
from jax import lax
from jax.experimental import pallas as pl
from jax.experimental.pallas import tpu as pltpu


def kernel(Q, K, V):
    b, q_len, h, d = Q.shape
    k_len = K.shape[1]
    h2 = h // 2
    scale = d ** -0.5

    my_x = lax.axis_index("x")
    my_y = lax.axis_index("y")

    Kt = jnp.transpose(K, (0, 2, 3, 1))
    Vt = jnp.transpose(V, (0, 2, 3, 1))
    Qt = jnp.transpose(Q, (0, 2, 1, 3))

    def partials_body(y_ref, q_ref, k_ref, v_ref, m_ref, l_ref, o_ref):
        del y_ref
        s_rows = []
        for hi in range(h2):
            q = q_ref[0, hi]
            kt = k_ref[0, hi]
            s_rows.append(lax.dot_general(
                q, kt, (((1,), (0,)), ((), ())),
                preferred_element_type=jnp.float32))
        s = jnp.concatenate(s_rows, axis=0) * scale
        m = jnp.max(s, axis=1, keepdims=True)
        p = jnp.exp(s - m)
        l = jnp.sum(p, axis=1, keepdims=True)
        o_rows = []
        for hi in range(h2):
            vt = v_ref[0, hi]
            o_rows.append(lax.dot_general(
                p[hi:hi + 1, :], vt, (((1,), (1,)), ((), ())),
                preferred_element_type=jnp.float32))
        o = jnp.concatenate(o_rows, axis=0)
        m_ref[...] = m.reshape(1, h2, 1)
        l_ref[...] = l.reshape(1, h2, 1)
        o_ref[...] = o.reshape(1, h2, d)

    y_arr = jnp.full((1,), my_y, dtype=jnp.int32)
    m_part, l_part, o_part = pl.pallas_call(
        partials_body,
        grid_spec=pltpu.PrefetchScalarGridSpec(
            num_scalar_prefetch=1,
            grid=(b,),
            in_specs=[
                pl.BlockSpec((1, h2, 1, d), lambda bi, y: (bi, y[0], 0, 0)),
                pl.BlockSpec((1, h2, d, k_len),
                             lambda bi, y: (bi, y[0], 0, 0)),
                pl.BlockSpec((1, h2, d, k_len),
                             lambda bi, y: (bi, y[0], 0, 0)),
            ],
            out_specs=[
                pl.BlockSpec((1, h2, 1), lambda bi, y: (bi, 0, 0)),
                pl.BlockSpec((1, h2, 1), lambda bi, y: (bi, 0, 0)),
                pl.BlockSpec((1, h2, d), lambda bi, y: (bi, 0, 0)),
            ],
        ),
        out_shape=[
            jax.ShapeDtypeStruct((b, h2, 1), jnp.float32),
            jax.ShapeDtypeStruct((b, h2, 1), jnp.float32),
            jax.ShapeDtypeStruct((b, h2, d), jnp.float32),
        ],
        compiler_params=pltpu.CompilerParams(
            vmem_limit_bytes=100 * 1024 * 1024,
        ),
    )(y_arr, Qt, Kt, Vt)

    def combine_body(m_ref, l_ref, o_ref, out_ref,
                     am, al, ao, rm, rl, ro, send_sems, recv_sems):
        x = lax.axis_index("x")
        y = lax.axis_index("y")
        x_nbr = (1 - x, y)
        y_nbr = (x, 1 - y)

        barrier = pltpu.get_barrier_semaphore()
        for nbr in (x_nbr, y_nbr):
            pl.semaphore_signal(
                barrier, inc=1, device_id=nbr,
                device_id_type=pl.DeviceIdType.MESH,
            )
        pl.semaphore_wait(barrier, 2)

        off = y * h2
        am[:, pl.ds(off, h2), :] = m_ref[...]
        al[:, pl.ds(off, h2), :] = l_ref[...]
        ao[:, pl.ds(off, h2), :] = o_ref[...]

        ph1 = []
        for i, (src, dst) in enumerate(
            [(m_ref, am), (l_ref, al), (o_ref, ao)]
        ):
            c = pltpu.make_async_remote_copy(
                src_ref=src,
                dst_ref=dst.at[:, pl.ds(off, h2), :],
                send_sem=send_sems.at[i],
                recv_sem=recv_sems.at[i],
                device_id=y_nbr,
                device_id_type=pl.DeviceIdType.MESH,
            )
            c.start()
            ph1.append(c)
        for c in ph1:
            c.wait()

        ph2 = []
        for i, (src, dst) in enumerate(
            [(am, rm), (al, rl), (ao, ro)], start=3
        ):
            c = pltpu.make_async_remote_copy(
                src_ref=src,
                dst_ref=dst,
                send_sem=send_sems.at[i],
                recv_sem=recv_sems.at[i],
                device_id=x_nbr,
                device_id_type=pl.DeviceIdType.MESH,
            )
            c.start()
            ph2.append(c)
        for c in ph2:
            c.wait()

        m_a = am[...]
        m_b = rm[...]
        m_n = jnp.maximum(m_a, m_b)
        a_a = jnp.exp(m_a - m_n)
        a_b = jnp.exp(m_b - m_n)
        l_n = a_a * al[...] + a_b * rl[...]
        o_n = (a_a * ao[...] + a_b * ro[...]) / l_n
        out_ref[:, 0, :, :] = o_n

    return pl.pallas_call(
        combine_body,
        in_specs=[
            pl.BlockSpec(memory_space=pltpu.VMEM),
            pl.BlockSpec(memory_space=pltpu.VMEM),
            pl.BlockSpec(memory_space=pltpu.VMEM),
        ],
        out_specs=pl.BlockSpec(memory_space=pltpu.VMEM),
        out_shape=jax.ShapeDtypeStruct((b, q_len, h, d), jnp.float32),
        scratch_shapes=[
            pltpu.VMEM((b, h, 1), jnp.float32),
            pltpu.VMEM((b, h, 1), jnp.float32),
            pltpu.VMEM((b, h, d), jnp.float32),
            pltpu.VMEM((b, h, 1), jnp.float32),
            pltpu.VMEM((b, h, 1), jnp.float32),
            pltpu.VMEM((b, h, d), jnp.float32),
            pltpu.SemaphoreType.DMA((6,)),
            pltpu.SemaphoreType.DMA((6,)),
        ],
        compiler_params=pltpu.CompilerParams(collective_id=0),
    )(m_part, l_part, o_part)


# device time: 37087 ns/iter; 8.8800x vs baseline; 1.1296x over previous
import jax
import jax.numpy as jnp
from jax import lax
from jax.experimental import pallas as pl
from jax.experimental.pallas import tpu as pltpu


def kernel(Q, K, V):
    b, q_len, h, d = Q.shape
    k_len = K.shape[1]
    h2 = h // 2
    scale = d ** -0.5

    my_y = lax.axis_index("y")

    Kt = jnp.transpose(K, (0, 2, 3, 1))
    Vt = jnp.transpose(V, (0, 2, 3, 1))
    Qt = jnp.transpose(Q, (0, 2, 1, 3))

    def body(y_ref, q_ref, k_ref, v_ref, out_ref,
             pm, pl_, po, am, al, ao, rm, rl, ro, send_sems, recv_sems):
        bi = pl.program_id(0)

        s_rows = []
        for hi in range(h2):
            q = q_ref[0, hi]
            kt = k_ref[0, hi]
            s_rows.append(lax.dot_general(
                q, kt, (((1,), (0,)), ((), ())),
                preferred_element_type=jnp.float32))
        s = jnp.concatenate(s_rows, axis=0) * scale
        m = jnp.max(s, axis=1, keepdims=True)
        p = jnp.exp(s - m)
        l = jnp.sum(p, axis=1, keepdims=True)
        o_rows = []
        for hi in range(h2):
            vt = v_ref[0, hi]
            o_rows.append(lax.dot_general(
                p[hi:hi + 1, :], vt, (((1,), (1,)), ((), ())),
                preferred_element_type=jnp.float32))
        o = jnp.concatenate(o_rows, axis=0)
        pm[pl.ds(bi, 1), :, :] = m.reshape(1, h2, 1)
        pl_[pl.ds(bi, 1), :, :] = l.reshape(1, h2, 1)
        po[pl.ds(bi, 1), :, :] = o.reshape(1, h2, d)

        @pl.when(bi == b - 1)
        def _():
            x = lax.axis_index("x")
            y = lax.axis_index("y")
            x_nbr = (1 - x, y)
            y_nbr = (x, 1 - y)
            diag = (1 - x, 1 - y)

            barrier = pltpu.get_barrier_semaphore()
            for nbr in (x_nbr, y_nbr, diag):
                pl.semaphore_signal(
                    barrier, inc=1, device_id=nbr,
                    device_id_type=pl.DeviceIdType.MESH,
                )
            pl.semaphore_wait(barrier, 3)

            off = y * h2
            am[:, pl.ds(off, h2), :] = pm[...]
            al[:, pl.ds(off, h2), :] = pl_[...]
            ao[:, pl.ds(off, h2), :] = po[...]

            copies = []
            targets = [
                (y_nbr, (am, al, ao)),
                (x_nbr, (rm, rl, ro)),
                (diag, (rm, rl, ro)),
            ]
            for t, (tgt, dsts) in enumerate(targets):
                for i, (src, dst) in enumerate(zip((pm, pl_, po), dsts)):
                    c = pltpu.make_async_remote_copy(
                        src_ref=src,
                        dst_ref=dst.at[:, pl.ds(off, h2), :],
                        send_sem=send_sems.at[3 * t + i],
                        recv_sem=recv_sems.at[3 * t + i],
                        device_id=tgt,
                        device_id_type=pl.DeviceIdType.MESH,
                    )
                    c.start()
                    copies.append(c)
            for c in copies:
                c.wait()

            m_a = am[...]
            m_b = rm[...]
            m_n = jnp.maximum(m_a, m_b)
            a_a = jnp.exp(m_a - m_n)
            a_b = jnp.exp(m_b - m_n)
            l_n = a_a * al[...] + a_b * rl[...]
            o_n = (a_a * ao[...] + a_b * ro[...]) / l_n
            out_ref[:, 0, :, :] = o_n

    y_arr = jnp.full((1,), my_y, dtype=jnp.int32)
    return pl.pallas_call(
        body,
        grid_spec=pltpu.PrefetchScalarGridSpec(
            num_scalar_prefetch=1,
            grid=(b,),
            in_specs=[
                pl.BlockSpec((1, h2, 1, d), lambda bi, y: (bi, y[0], 0, 0)),
                pl.BlockSpec((1, h2, d, k_len),
                             lambda bi, y: (bi, y[0], 0, 0)),
                pl.BlockSpec((1, h2, d, k_len),
                             lambda bi, y: (bi, y[0], 0, 0)),
            ],
            out_specs=pl.BlockSpec((b, q_len, h, d),
                                   lambda bi, y: (0, 0, 0, 0)),
            scratch_shapes=[
                pltpu.VMEM((b, h2, 1), jnp.float32),
                pltpu.VMEM((b, h2, 1), jnp.float32),
                pltpu.VMEM((b, h2, d), jnp.float32),
                pltpu.VMEM((b, h, 1), jnp.float32),
                pltpu.VMEM((b, h, 1), jnp.float32),
                pltpu.VMEM((b, h, d), jnp.float32),
                pltpu.VMEM((b, h, 1), jnp.float32),
                pltpu.VMEM((b, h, 1), jnp.float32),
                pltpu.VMEM((b, h, d), jnp.float32),
                pltpu.SemaphoreType.DMA((9,)),
                pltpu.SemaphoreType.DMA((9,)),
            ],
        ),
        out_shape=jax.ShapeDtypeStruct((b, q_len, h, d), jnp.float32),
        compiler_params=pltpu.CompilerParams(
            collective_id=0,
            vmem_limit_bytes=100 * 1024 * 1024,
        ),
    )(y_arr, Qt, Kt, Vt)


# device time: 33938 ns/iter; 9.7040x vs baseline; 1.0928x over previous
import jax
import jax.numpy as jnp
from jax import lax
from jax.experimental import pallas as pl
from jax.experimental.pallas import tpu as pltpu

BB = 2


def kernel(Q, K, V):
    b, q_len, h, d = Q.shape
    k_len = K.shape[1]
    h2 = h // 2
    scale = d ** -0.5
    n_steps = b // BB

    my_y = lax.axis_index("y")

    Kt = jnp.transpose(K, (0, 2, 3, 1))
    Vt = jnp.transpose(V, (0, 2, 3, 1))

    def body(y_ref, q_ref, k_ref, v_ref, out_ref,
             pm, pl_, po, am, al, ao, rm, rl, ro, send_sems, recv_sems):
        bi = pl.program_id(0)
        off = y_ref[0] * h2

        for bj in range(BB):
            s_rows = []
            for hi in range(h2):
                q = q_ref[bj, 0, pl.ds(off + hi, 1), :]
                kt = k_ref[bj, hi]
                s_rows.append(lax.dot_general(
                    q, kt, (((1,), (0,)), ((), ())),
                    preferred_element_type=jnp.float32))
            s = jnp.concatenate(s_rows, axis=0) * scale
            m = jnp.max(s, axis=1, keepdims=True)
            p = jnp.exp(s - m)
            l = jnp.sum(p, axis=1, keepdims=True)
            o_rows = []
            for hi in range(h2):
                vt = v_ref[bj, hi]
                o_rows.append(lax.dot_general(
                    p[hi:hi + 1, :], vt, (((1,), (1,)), ((), ())),
                    preferred_element_type=jnp.float32))
            o = jnp.concatenate(o_rows, axis=0)
            bg = BB * bi + bj
            pm[pl.ds(bg, 1), :, :] = m.reshape(1, h2, 1)
            pl_[pl.ds(bg, 1), :, :] = l.reshape(1, h2, 1)
            po[pl.ds(bg, 1), :, :] = o.reshape(1, h2, d)

        @pl.when(bi == n_steps - 1)
        def _():
            x = lax.axis_index("x")
            y = lax.axis_index("y")
            x_nbr = (1 - x, y)
            y_nbr = (x, 1 - y)
            diag = (1 - x, 1 - y)

            barrier = pltpu.get_barrier_semaphore()
            for nbr in (x_nbr, y_nbr, diag):
                pl.semaphore_signal(
                    barrier, inc=1, device_id=nbr,
                    device_id_type=pl.DeviceIdType.MESH,
                )
            pl.semaphore_wait(barrier, 3)

            am[:, pl.ds(off, h2), :] = pm[...]
            al[:, pl.ds(off, h2), :] = pl_[...]
            ao[:, pl.ds(off, h2), :] = po[...]

            copies = []
            targets = [
                (y_nbr, (am, al, ao)),
                (x_nbr, (rm, rl, ro)),
                (diag, (rm, rl, ro)),
            ]
            for t, (tgt, dsts) in enumerate(targets):
                for i, (src, dst) in enumerate(zip((pm, pl_, po), dsts)):
                    c = pltpu.make_async_remote_copy(
                        src_ref=src,
                        dst_ref=dst.at[:, pl.ds(off, h2), :],
                        send_sem=send_sems.at[3 * t + i],
                        recv_sem=recv_sems.at[3 * t + i],
                        device_id=tgt,
                        device_id_type=pl.DeviceIdType.MESH,
                    )
                    c.start()
                    copies.append(c)
            for c in copies:
                c.wait()

            m_a = am[...]
            m_b = rm[...]
            m_n = jnp.maximum(m_a, m_b)
            a_a = jnp.exp(m_a - m_n)
            a_b = jnp.exp(m_b - m_n)
            l_n = a_a * al[...] + a_b * rl[...]
            o_n = (a_a * ao[...] + a_b * ro[...]) / l_n
            out_ref[:, 0, :, :] = o_n

    y_arr = jnp.full((1,), my_y, dtype=jnp.int32)
    return pl.pallas_call(
        body,
        grid_spec=pltpu.PrefetchScalarGridSpec(
            num_scalar_prefetch=1,
            grid=(n_steps,),
            in_specs=[
                pl.BlockSpec((BB, 1, h, d), lambda bi, y: (bi, 0, 0, 0)),
                pl.BlockSpec((BB, h2, d, k_len),
                             lambda bi, y: (bi, y[0], 0, 0)),
                pl.BlockSpec((BB, h2, d, k_len),
                             lambda bi, y: (bi, y[0], 0, 0)),
            ],
            out_specs=pl.BlockSpec((b, q_len, h, d),
                                   lambda bi, y: (0, 0, 0, 0)),
            scratch_shapes=[
                pltpu.VMEM((b, h2, 1), jnp.float32),
                pltpu.VMEM((b, h2, 1), jnp.float32),
                pltpu.VMEM((b, h2, d), jnp.float32),
                pltpu.VMEM((b, h, 1), jnp.float32),
                pltpu.VMEM((b, h, 1), jnp.float32),
                pltpu.VMEM((b, h, d), jnp.float32),
                pltpu.VMEM((b, h, 1), jnp.float32),
                pltpu.VMEM((b, h, 1), jnp.float32),
                pltpu.VMEM((b, h, d), jnp.float32),
                pltpu.SemaphoreType.DMA((9,)),
                pltpu.SemaphoreType.DMA((9,)),
            ],
        ),
        out_shape=jax.ShapeDtypeStruct((b, q_len, h, d), jnp.float32),
        compiler_params=pltpu.CompilerParams(
            collective_id=0,
            vmem_limit_bytes=100 * 1024 * 1024,
        ),
    )(y_arr, Q, Kt, Vt)
